# baseline (speedup 1.0000x reference)
import functools

import jax
import jax.numpy as jnp
from jax.experimental import pallas as pl
from jax.experimental.pallas import tpu as pltpu


_NSPLIT = 2


def _fused_body(x_ref, w1_ref, b1_ref, w2_ref, b2_ref, w3_ref, b3_ref,
                wc_ref, d1_ref, c1_ref, d2_ref, c2_ref, d3_ref, c3_ref,
                xr_ref, ze_ref, emb_ref):
    wc = wc_ref[...]
    cnorm = jnp.sum(wc * wc, axis=0, keepdims=True)
    wcm2 = wc * -2.0
    bb = x_ref.shape[0]
    sb = bb // _NSPLIT
    for s in range(_NSPLIT):
        rows = pl.ds(s * sb, sb)
        h = jnp.dot(x_ref[rows, :], w1_ref[...],
                    preferred_element_type=jnp.float32) + b1_ref[...]
        h = jnp.maximum(h, 0.0).astype(jnp.bfloat16)
        h = jnp.dot(h, w2_ref[...], preferred_element_type=jnp.float32) + b2_ref[...]
        h = jnp.maximum(h, 0.0)
        ze = jnp.dot(h, w3_ref[...], preferred_element_type=jnp.float32) + b3_ref[...]
        ze_ref[rows, :] = ze

        dist = jnp.dot(ze, wcm2, preferred_element_type=jnp.float32) + cnorm
        idx = jnp.argmin(dist, axis=1)
        onehot = (jax.lax.broadcasted_iota(jnp.int32, dist.shape, 1)
                  == idx[:, None]).astype(jnp.float32)
        emb = jax.lax.dot_general(onehot, wc, (((1,), (1,)), ((), ())),
                                  preferred_element_type=jnp.float32)
        emb_ref[rows, :] = emb

        h = jnp.dot(emb, d1_ref[...], preferred_element_type=jnp.float32) + c1_ref[...]
        h = jnp.maximum(h, 0.0).astype(jnp.bfloat16)
        h = jnp.dot(h, d2_ref[...], preferred_element_type=jnp.float32) + c2_ref[...]
        h = jnp.maximum(h, 0.0)
        xr_ref[rows, :] = (jnp.dot(h, d3_ref[...],
                                   preferred_element_type=jnp.float32) + c3_ref[...])


@functools.partial(jax.jit, static_argnames=("bb",))
def _run(x, W1, b1, W2, b2, W3, b3, Wc, D1, c1, D2, c2, D3, c3, bb=4096):
    B, IN = x.shape
    HID = W1.shape[1]
    HALF = W2.shape[1]
    EMB = W3.shape[1]
    K = Wc.shape[1]
    grid = (B // bb,)

    def full(a):
        return pl.BlockSpec(a.shape, lambda i: (0,) * a.ndim)

    b1r, b2r, b3r = b1[None, :], b2[None, :], b3[None, :]
    c1r, c2r, c3r = c1[None, :], c2[None, :], c3[None, :]
    W2 = W2.astype(jnp.bfloat16)
    D2 = D2.astype(jnp.bfloat16)

    batch_spec = pl.BlockSpec((bb, IN), lambda i: (i, 0))
    out_shapes = (
        jax.ShapeDtypeStruct((B, IN), jnp.float32),
        jax.ShapeDtypeStruct((B, EMB), jnp.float32),
        jax.ShapeDtypeStruct((B, EMB), jnp.float32),
    )
    out_specs = (
        pl.BlockSpec((bb, IN), lambda i: (i, 0)),
        pl.BlockSpec((bb, EMB), lambda i: (i, 0)),
        pl.BlockSpec((bb, EMB), lambda i: (i, 0)),
    )
    in_specs = [batch_spec] + [full(a) for a in
                               (W1, b1r, W2, b2r, W3, b3r, Wc,
                                D1, c1r, D2, c2r, D3, c3r)]
    return pl.pallas_call(
        _fused_body,
        grid=grid,
        in_specs=in_specs,
        out_specs=out_specs,
        out_shape=out_shapes,
        compiler_params=pltpu.CompilerParams(
            dimension_semantics=("parallel",)),
    )(x, W1, b1r, W2, b2r, W3, b3r, Wc, D1, c1r, D2, c2r, D3, c3r)


def kernel(x, W1, b1, W2, b2, W3, b3, Wc, D1, c1, D2, c2, D3, c3):
    x_recon, z_e, emb = _run(x, W1, b1, W2, b2, W3, b3, Wc,
                             D1, c1, D2, c2, D3, c3)
    return (x_recon, z_e, emb)


# speedup vs baseline: 1.2613x; 1.2613x over previous
import functools

import jax
import jax.numpy as jnp
from jax.experimental import pallas as pl
from jax.experimental.pallas import tpu as pltpu


_NSPLIT = 2


def _fused_body(x_ref, w1_ref, b1_ref, w2_ref, b2_ref, w3_ref, b3_ref,
                wc_ref, d1_ref, c1_ref, d2_ref, c2_ref, d3_ref, c3_ref,
                xr_ref, ze_ref, emb_ref):
    wc = wc_ref[...]
    cnorm = jnp.sum(wc * wc, axis=0, keepdims=True)
    wcm2 = wc * -2.0
    bb = x_ref.shape[0]
    sb = bb // _NSPLIT
    rows = [pl.ds(s * sb, sb) for s in range(_NSPLIT)]

    def enc1(r):
        h = jnp.dot(x_ref[r, :], w1_ref[...],
                    preferred_element_type=jnp.float32) + b1_ref[...]
        return jnp.maximum(h, 0.0)

    def enc2(h):
        h = jnp.dot(h, w2_ref[...], preferred_element_type=jnp.float32) + b2_ref[...]
        return jnp.maximum(h, 0.0)

    def enc3(h, r):
        ze = jnp.dot(h, w3_ref[...], preferred_element_type=jnp.float32) + b3_ref[...]
        ze_ref[r, :] = ze
        return ze

    def quant(ze, r):
        dist = jnp.dot(ze, wcm2, preferred_element_type=jnp.float32) + cnorm
        idx = jnp.argmin(dist, axis=1)
        onehot = (jax.lax.broadcasted_iota(jnp.int32, dist.shape, 1)
                  == idx[:, None]).astype(jnp.float32)
        emb = jax.lax.dot_general(onehot, wc, (((1,), (1,)), ((), ())),
                                  preferred_element_type=jnp.float32)
        emb_ref[r, :] = emb
        return emb

    def dec1(emb):
        h = jnp.dot(emb, d1_ref[...], preferred_element_type=jnp.float32) + c1_ref[...]
        return jnp.maximum(h, 0.0)

    def dec2(h):
        h = jnp.dot(h, d2_ref[...], preferred_element_type=jnp.float32) + c2_ref[...]
        return jnp.maximum(h, 0.0)

    def dec3(h, r):
        xr_ref[r, :] = (jnp.dot(h, d3_ref[...],
                                preferred_element_type=jnp.float32) + c3_ref[...])

    ha = enc1(rows[0])
    hb = enc1(rows[1])
    ha = enc2(ha)
    hb = enc2(hb)
    za = enc3(ha, rows[0])
    zb = enc3(hb, rows[1])
    ea = quant(za, rows[0])
    eb = quant(zb, rows[1])
    ha = dec1(ea)
    hb = dec1(eb)
    ha = dec2(ha)
    hb = dec2(hb)
    dec3(ha, rows[0])
    dec3(hb, rows[1])


@functools.partial(jax.jit, static_argnames=("bb",))
def _run(x, W1, b1, W2, b2, W3, b3, Wc, D1, c1, D2, c2, D3, c3, bb=4096):
    B, IN = x.shape
    HID = W1.shape[1]
    HALF = W2.shape[1]
    EMB = W3.shape[1]
    K = Wc.shape[1]
    grid = (B // bb,)

    def full(a):
        return pl.BlockSpec(a.shape, lambda i: (0,) * a.ndim)

    b1r, b2r, b3r = b1[None, :], b2[None, :], b3[None, :]
    c1r, c2r, c3r = c1[None, :], c2[None, :], c3[None, :]

    batch_spec = pl.BlockSpec((bb, IN), lambda i: (i, 0))
    out_shapes = (
        jax.ShapeDtypeStruct((B, IN), jnp.float32),
        jax.ShapeDtypeStruct((B, EMB), jnp.float32),
        jax.ShapeDtypeStruct((B, EMB), jnp.float32),
    )
    out_specs = (
        pl.BlockSpec((bb, IN), lambda i: (i, 0)),
        pl.BlockSpec((bb, EMB), lambda i: (i, 0)),
        pl.BlockSpec((bb, EMB), lambda i: (i, 0)),
    )
    in_specs = [batch_spec] + [full(a) for a in
                               (W1, b1r, W2, b2r, W3, b3r, Wc,
                                D1, c1r, D2, c2r, D3, c3r)]
    return pl.pallas_call(
        _fused_body,
        grid=grid,
        in_specs=in_specs,
        out_specs=out_specs,
        out_shape=out_shapes,
        compiler_params=pltpu.CompilerParams(
            dimension_semantics=("parallel",)),
    )(x, W1, b1r, W2, b2r, W3, b3r, Wc, D1, c1r, D2, c2r, D3, c3r)


def kernel(x, W1, b1, W2, b2, W3, b3, Wc, D1, c1, D2, c2, D3, c3):
    x_recon, z_e, emb = _run(x, W1, b1, W2, b2, W3, b3, Wc,
                             D1, c1, D2, c2, D3, c3)
    return (x_recon, z_e, emb)


# speedup vs baseline: 1.3142x; 1.0419x over previous
import functools

import jax
import jax.numpy as jnp
from jax.experimental import pallas as pl
from jax.experimental.pallas import tpu as pltpu


_NSPLIT = 16


def _fused_body(x_ref, w1_ref, b1_ref, w2_ref, b2_ref, w3_ref, b3_ref,
                wc_ref, d1_ref, c1_ref, d2_ref, c2_ref, d3_ref, c3_ref,
                xr_ref, ze_ref, emb_ref):
    wc = wc_ref[...]
    cnorm = jnp.sum(wc * wc, axis=0, keepdims=True)
    wcm2 = wc * -2.0
    bb = x_ref.shape[0]
    sb = bb // _NSPLIT
    rows = [pl.ds(s * sb, sb) for s in range(_NSPLIT)]

    def enc1(r):
        h = jnp.dot(x_ref[r, :], w1_ref[...],
                    preferred_element_type=jnp.float32) + b1_ref[...]
        return jnp.maximum(h, 0.0)

    def enc2(h):
        h = jnp.dot(h, w2_ref[...], preferred_element_type=jnp.float32) + b2_ref[...]
        return jnp.maximum(h, 0.0)

    def enc3(h, r):
        ze = jnp.dot(h, w3_ref[...], preferred_element_type=jnp.float32) + b3_ref[...]
        ze_ref[r, :] = ze
        return ze

    def quant(ze, r):
        dist = jnp.dot(ze, wcm2, preferred_element_type=jnp.float32) + cnorm
        idx = jnp.argmin(dist, axis=1)
        onehot = (jax.lax.broadcasted_iota(jnp.int32, dist.shape, 1)
                  == idx[:, None]).astype(jnp.float32)
        emb = jax.lax.dot_general(onehot, wc, (((1,), (1,)), ((), ())),
                                  preferred_element_type=jnp.float32)
        emb_ref[r, :] = emb
        return emb

    def dec1(emb):
        h = jnp.dot(emb, d1_ref[...], preferred_element_type=jnp.float32) + c1_ref[...]
        return jnp.maximum(h, 0.0)

    def dec2(h):
        h = jnp.dot(h, d2_ref[...], preferred_element_type=jnp.float32) + c2_ref[...]
        return jnp.maximum(h, 0.0)

    def dec3(h, r):
        xr_ref[r, :] = (jnp.dot(h, d3_ref[...],
                                preferred_element_type=jnp.float32) + c3_ref[...])

    hs = [enc1(r) for r in rows]
    hs = [enc2(h) for h in hs]
    zs = [enc3(h, r) for h, r in zip(hs, rows)]
    es = [quant(z, r) for z, r in zip(zs, rows)]
    hs = [dec1(e) for e in es]
    hs = [dec2(h) for h in hs]
    for h, r in zip(hs, rows):
        dec3(h, r)


@functools.partial(jax.jit, static_argnames=("bb",))
def _run(x, W1, b1, W2, b2, W3, b3, Wc, D1, c1, D2, c2, D3, c3, bb=4096):
    B, IN = x.shape
    HID = W1.shape[1]
    HALF = W2.shape[1]
    EMB = W3.shape[1]
    K = Wc.shape[1]
    grid = (B // bb,)

    def full(a):
        return pl.BlockSpec(a.shape, lambda i: (0,) * a.ndim)

    b1r, b2r, b3r = b1[None, :], b2[None, :], b3[None, :]
    c1r, c2r, c3r = c1[None, :], c2[None, :], c3[None, :]

    batch_spec = pl.BlockSpec((bb, IN), lambda i: (i, 0))
    out_shapes = (
        jax.ShapeDtypeStruct((B, IN), jnp.float32),
        jax.ShapeDtypeStruct((B, EMB), jnp.float32),
        jax.ShapeDtypeStruct((B, EMB), jnp.float32),
    )
    out_specs = (
        pl.BlockSpec((bb, IN), lambda i: (i, 0)),
        pl.BlockSpec((bb, EMB), lambda i: (i, 0)),
        pl.BlockSpec((bb, EMB), lambda i: (i, 0)),
    )
    in_specs = [batch_spec] + [full(a) for a in
                               (W1, b1r, W2, b2r, W3, b3r, Wc,
                                D1, c1r, D2, c2r, D3, c3r)]
    return pl.pallas_call(
        _fused_body,
        grid=grid,
        in_specs=in_specs,
        out_specs=out_specs,
        out_shape=out_shapes,
        compiler_params=pltpu.CompilerParams(
            dimension_semantics=("parallel",)),
    )(x, W1, b1r, W2, b2r, W3, b3r, Wc, D1, c1r, D2, c2r, D3, c3r)


def kernel(x, W1, b1, W2, b2, W3, b3, Wc, D1, c1, D2, c2, D3, c3):
    x_recon, z_e, emb = _run(x, W1, b1, W2, b2, W3, b3, Wc,
                             D1, c1, D2, c2, D3, c3)
    return (x_recon, z_e, emb)
